# Initial kernel scaffold; baseline (speedup 1.0000x reference)
import jax
import jax.numpy as jnp
from jax.experimental import pallas as pl


def kernel(params, lits_var, lits_sign):
    raise NotImplementedError("write your pallas kernel here")


# speedup vs baseline: 3.4720x; 3.4720x over previous
import functools

import jax
import jax.numpy as jnp
from jax import lax
from jax.experimental import pallas as pl
from jax.experimental.pallas import tpu as pltpu
from jax.experimental.pallas import tpu_sc as plsc

NV = 10000
NC = 42000
KL = 3
F = 128
Q = 32

NC_PAD = 45056
C_BLK = 2048
C_GRID = NC_PAD // C_BLK
B_PAD = KL * NC_PAD
V_BLK = 2000
V_GRID = NV // V_BLK

SC_CORES = 2
SC_SUB = 16
SC_W = SC_CORES * SC_SUB
PER_W = B_PAD // SC_W
CHUNK = 128
N_CH = PER_W // CHUNK
NV_PAD = 10240
ZROWS = NV_PAD // SC_SUB

_F32 = jnp.float32


def _relu(x):
    return jnp.maximum(x, 0.0)


def _softplus(x):
    return jnp.maximum(x, 0.0) + jnp.log1p(jnp.exp(-jnp.abs(x)))


def _sigmoid(x):
    return 1.0 / (1.0 + jnp.exp(-x))


def _const_state_row():
    lane = lax.broadcasted_iota(jnp.int32, (1, F), 1)
    scale = jnp.float32(jnp.sqrt(128.0) * 0.25)
    hi = jnp.float32(1.0 - 1.0 / 128.0) * scale
    lo = jnp.float32(-1.0 / 128.0) * scale
    return jnp.where(lane == 0, hi, lo)


def _full_spec(shape):
    nd = len(shape)
    return pl.BlockSpec(shape, lambda *_: (0,) * nd)


def _qmlp(vars_in, noise, w1v, w1n, b1, w2, b2, w3, b3):
    has_v = vars_in is not None

    def body(*refs):
        if has_v:
            v_r, n_r, w1v_r, w1n_r, b1_r, w2_r, b2_r, w3_r, b3_r, o_r = refs
            xv = v_r[...]
        else:
            n_r, w1v_r, w1n_r, b1_r, w2_r, b2_r, w3_r, b3_r, o_r = refs
            xv = _const_state_row()
        h = jnp.dot(xv, w1v_r[...], preferred_element_type=_F32)
        h = h + jnp.dot(n_r[...], w1n_r[...], preferred_element_type=_F32)
        h = _relu(h + b1_r[...])
        h = _relu(jnp.dot(h, w2_r[...], preferred_element_type=_F32) + b2_r[...])
        o_r[...] = jnp.dot(h, w3_r[...], preferred_element_type=_F32) + b3_r[...]

    in_specs = []
    args = []
    if has_v:
        in_specs.append(pl.BlockSpec((V_BLK, F), lambda i: (i, 0)))
        args.append(vars_in)
    in_specs += [pl.BlockSpec((V_BLK, 4), lambda i: (i, 0))]
    args += [noise]
    for w in (w1v, w1n, b1, w2, b2, w3, b3):
        in_specs.append(_full_spec(w.shape))
        args.append(w)
    return pl.pallas_call(
        body,
        grid=(V_GRID,),
        in_specs=in_specs,
        out_specs=pl.BlockSpec((V_BLK, Q), lambda i: (i, 0)),
        out_shape=jax.ShapeDtypeStruct((NV, Q), _F32),
    )(*args)


def _edge_pre(g3, sign3):
    def body(g_r, s_r, cl_r, co_r):
        t = []
        sp_sum = jnp.zeros((C_BLK, Q), _F32)
        for k in range(KL):
            tk = g_r[k] * s_r[k]
            t.append(tk)
            sp_sum = sp_sum + _softplus(tk)
        cl = jnp.exp(-sp_sum)
        cl_r[...] = cl
        for k in range(KL):
            co_r[k] = -cl * _sigmoid(t[k]) * s_r[k]

    return pl.pallas_call(
        body,
        grid=(C_GRID,),
        in_specs=[
            pl.BlockSpec((KL, C_BLK, Q), lambda i: (0, i, 0)),
            pl.BlockSpec((KL, C_BLK, 1), lambda i: (0, i, 0)),
        ],
        out_specs=[
            pl.BlockSpec((C_BLK, Q), lambda i: (i, 0)),
            pl.BlockSpec((KL, C_BLK, Q), lambda i: (0, i, 0)),
        ],
        out_shape=[
            jax.ShapeDtypeStruct((NC_PAD, Q), _F32),
            jax.ShapeDtypeStruct((KL, NC_PAD, Q), _F32),
        ],
    )(g3, sign3)


def _clause_mlp_r0(cl, posw, negw, w1a, w1b, b1, w2, b2, w3v, b3v, w3n, b3n):
    def body(cl_r, pw_r, nw_r, w1a_r, w1b_r, b1_r, w2_r, b2_r, w3v_r, b3v_r,
             w3n_r, b3n_r, evl_r, ncv_r, ps_r, pq_r):
        i = pl.program_id(0)
        crow = _const_state_row()
        h = jnp.dot(crow, w1a_r[...], preferred_element_type=_F32)
        h = h + jnp.dot(cl_r[...], w1b_r[...], preferred_element_type=_F32)
        h = _relu(h + b1_r[...])
        h = _relu(jnp.dot(h, w2_r[...], preferred_element_type=_F32) + b2_r[...])
        vla = jnp.dot(h, w3v_r[...], preferred_element_type=_F32) + b3v_r[...]
        ncv = jnp.dot(h, w3n_r[...], preferred_element_type=_F32) + b3n_r[...]
        ncv_r[...] = ncv
        for k in range(KL):
            evl_r[k] = jnp.concatenate([vla * pw_r[k], vla * nw_r[k]], axis=1)
        rows = i * C_BLK + lax.broadcasted_iota(jnp.int32, (C_BLK, 1), 0)
        m = (rows < NC).astype(_F32)
        ncv_m = ncv * m
        ps_r[...] = jnp.sum(ncv_m, axis=0).reshape(1, 1, F)
        pq_r[...] = jnp.zeros((1, 1, F), _F32) + jnp.sum(ncv_m * ncv_m)

    return pl.pallas_call(
        body,
        grid=(C_GRID,),
        in_specs=[
            pl.BlockSpec((C_BLK, Q), lambda i: (i, 0)),
            pl.BlockSpec((KL, C_BLK, 1), lambda i: (0, i, 0)),
            pl.BlockSpec((KL, C_BLK, 1), lambda i: (0, i, 0)),
            _full_spec(w1a.shape), _full_spec(w1b.shape), _full_spec(b1.shape),
            _full_spec(w2.shape), _full_spec(b2.shape),
            _full_spec(w3v.shape), _full_spec(b3v.shape),
            _full_spec(w3n.shape), _full_spec(b3n.shape),
        ],
        out_specs=[
            pl.BlockSpec((KL, C_BLK, 2 * Q), lambda i: (0, i, 0)),
            pl.BlockSpec((C_BLK, F), lambda i: (i, 0)),
            pl.BlockSpec((1, 1, F), lambda i: (i, 0, 0)),
            pl.BlockSpec((1, 1, F), lambda i: (i, 0, 0)),
        ],
        out_shape=[
            jax.ShapeDtypeStruct((KL, NC_PAD, 2 * Q), _F32),
            jax.ShapeDtypeStruct((NC_PAD, F), _F32),
            jax.ShapeDtypeStruct((C_GRID, 1, F), _F32),
            jax.ShapeDtypeStruct((C_GRID, 1, F), _F32),
        ],
    )(cl, posw, negw, w1a, w1b, b1, w2, b2, w3v, b3v, w3n, b3n)


def _clause_mlp_r1(cl, ncv0, ps0, pq0, posw, negw,
                   w1a, w1b, b1, w2, b2, w3v, b3v):
    def body(cl_r, ncv_r, ps_r, pq_r, pw_r, nw_r, w1a_r, w1b_r, b1_r,
             w2_r, b2_r, w3v_r, b3v_r, evl_r):
        mu = jnp.sum(ps_r[...], axis=0).reshape(1, F) / jnp.float32(NC)
        m2 = jnp.sum(mu * mu)
        tot = jnp.sum(pq_r[...]) / jnp.float32(F)
        scale = lax.rsqrt((tot - jnp.float32(NC) * m2) / jnp.float32(NC)
                          + jnp.float32(1e-6))
        crow = _const_state_row()
        cs = (ncv_r[...] - mu) * (scale * 0.25) + 0.1 * crow
        h = jnp.dot(cs, w1a_r[...], preferred_element_type=_F32)
        h = h + jnp.dot(cl_r[...], w1b_r[...], preferred_element_type=_F32)
        h = _relu(h + b1_r[...])
        h = _relu(jnp.dot(h, w2_r[...], preferred_element_type=_F32) + b2_r[...])
        vla = jnp.dot(h, w3v_r[...], preferred_element_type=_F32) + b3v_r[...]
        for k in range(KL):
            evl_r[k] = jnp.concatenate([vla * pw_r[k], vla * nw_r[k]], axis=1)

    return pl.pallas_call(
        body,
        grid=(C_GRID,),
        in_specs=[
            pl.BlockSpec((C_BLK, Q), lambda i: (i, 0)),
            pl.BlockSpec((C_BLK, F), lambda i: (i, 0)),
            _full_spec(ps0.shape), _full_spec(pq0.shape),
            pl.BlockSpec((KL, C_BLK, 1), lambda i: (0, i, 0)),
            pl.BlockSpec((KL, C_BLK, 1), lambda i: (0, i, 0)),
            _full_spec(w1a.shape), _full_spec(w1b.shape), _full_spec(b1.shape),
            _full_spec(w2.shape), _full_spec(b2.shape),
            _full_spec(w3v.shape), _full_spec(b3v.shape),
        ],
        out_specs=pl.BlockSpec((KL, C_BLK, 2 * Q), lambda i: (0, i, 0)),
        out_shape=jax.ShapeDtypeStruct((KL, NC_PAD, 2 * Q), _F32),
    )(cl, ncv0, ps0, pq0, posw, negw, w1a, w1b, b1, w2, b2, w3v, b3v)


def _update_mlp(vars_in, g0, g1, vl0, vl1, w1v, w1g, w1pn, b1, w2, b2, w3, b3):
    has_v = vars_in is not None

    def body(*refs):
        if has_v:
            (v_r, g0_r, g1_r, vl0_r, vl1_r, w1v_r, w1g_r, w1pn_r, b1_r,
             w2_r, b2_r, w3_r, b3_r, y_r, ps_r, pq_r) = refs
            xv = v_r[...]
        else:
            (g0_r, g1_r, vl0_r, vl1_r, w1v_r, w1g_r, w1pn_r, b1_r,
             w2_r, b2_r, w3_r, b3_r, y_r, ps_r, pq_r) = refs
            xv = _const_state_row()
        grad = g0_r[...] + g1_r[...]
        vl = vl0_r[...] + vl1_r[...]
        h = jnp.dot(xv, w1v_r[...], preferred_element_type=_F32)
        h = h + jnp.dot(grad, w1g_r[...], preferred_element_type=_F32)
        h = h + jnp.dot(vl, w1pn_r[...], preferred_element_type=_F32)
        h = _relu(h + b1_r[...])
        h = _relu(jnp.dot(h, w2_r[...], preferred_element_type=_F32) + b2_r[...])
        y = jnp.dot(h, w3_r[...], preferred_element_type=_F32) + b3_r[...]
        y_r[...] = y
        ps_r[...] = jnp.sum(y, axis=0).reshape(1, 1, F)
        pq_r[...] = jnp.zeros((1, 1, F), _F32) + jnp.sum(y * y)

    in_specs = []
    args = []
    if has_v:
        in_specs.append(pl.BlockSpec((V_BLK, F), lambda i: (i, 0)))
        args.append(vars_in)
    in_specs += [
        pl.BlockSpec((V_BLK, Q), lambda i: (i, 0)),
        pl.BlockSpec((V_BLK, Q), lambda i: (i, 0)),
        pl.BlockSpec((V_BLK, 2 * Q), lambda i: (i, 0)),
        pl.BlockSpec((V_BLK, 2 * Q), lambda i: (i, 0)),
    ]
    args += [g0, g1, vl0, vl1]
    for w in (w1v, w1g, w1pn, b1, w2, b2, w3, b3):
        in_specs.append(_full_spec(w.shape))
        args.append(w)
    return pl.pallas_call(
        body,
        grid=(V_GRID,),
        in_specs=in_specs,
        out_specs=[
            pl.BlockSpec((V_BLK, F), lambda i: (i, 0)),
            pl.BlockSpec((1, 1, F), lambda i: (i, 0, 0)),
            pl.BlockSpec((1, 1, F), lambda i: (i, 0, 0)),
        ],
        out_shape=[
            jax.ShapeDtypeStruct((NV, F), _F32),
            jax.ShapeDtypeStruct((V_GRID, 1, F), _F32),
            jax.ShapeDtypeStruct((V_GRID, 1, F), _F32),
        ],
    )(*args)


def _apply_logits(y, prev, ps, pq, w1, b1, w2, b2, w3, b3):
    has_p = prev is not None

    def body(*refs):
        if has_p:
            (y_r, p_r, ps_r, pq_r, w1_r, b1_r, w2_r, b2_r, w3_r, b3_r,
             nv_r, lg_r, lt_r) = refs
            pv = p_r[...]
        else:
            (y_r, ps_r, pq_r, w1_r, b1_r, w2_r, b2_r, w3_r, b3_r,
             nv_r, lg_r, lt_r) = refs
            pv = _const_state_row()
        mu = jnp.sum(ps_r[...], axis=0).reshape(1, F) / jnp.float32(NV)
        m2 = jnp.sum(mu * mu)
        tot = jnp.sum(pq_r[...]) / jnp.float32(F)
        scale = lax.rsqrt((tot - jnp.float32(NV) * m2) / jnp.float32(NV)
                          + jnp.float32(1e-6))
        nv = (y_r[...] - mu) * (scale * 0.25) + 0.1 * pv
        nv_r[...] = nv
        h = _relu(jnp.dot(nv, w1_r[...], preferred_element_type=_F32) + b1_r[...])
        h = _relu(jnp.dot(h, w2_r[...], preferred_element_type=_F32) + b2_r[...])
        lg = jnp.dot(h, w3_r[...], preferred_element_type=_F32) + b3_r[...]
        lg_r[...] = lg
        lt_r[...] = jnp.broadcast_to(lg, (V_BLK, 16))

    in_specs = [pl.BlockSpec((V_BLK, F), lambda i: (i, 0))]
    args = [y]
    if has_p:
        in_specs.append(pl.BlockSpec((V_BLK, F), lambda i: (i, 0)))
        args.append(prev)
    in_specs += [_full_spec(ps.shape), _full_spec(pq.shape)]
    args += [ps, pq]
    for w in (w1, b1, w2, b2, w3, b3):
        in_specs.append(_full_spec(w.shape))
        args.append(w)
    return pl.pallas_call(
        body,
        grid=(V_GRID,),
        in_specs=in_specs,
        out_specs=[
            pl.BlockSpec((V_BLK, F), lambda i: (i, 0)),
            pl.BlockSpec((V_BLK, 1), lambda i: (i, 0)),
            pl.BlockSpec((V_BLK, 16), lambda i: (i, 0)),
        ],
        out_shape=[
            jax.ShapeDtypeStruct((NV, F), _F32),
            jax.ShapeDtypeStruct((NV, 1), _F32),
            jax.ShapeDtypeStruct((NV, 16), _F32),
        ],
    )(*args)


def _loss_reduce(lg3, sign3):
    def body(g_r, s_r, o_r):
        i = pl.program_id(0)
        sp_sum = jnp.zeros((C_BLK, 1), _F32)
        for k in range(KL):
            t = g_r[k][:, 0:1] * s_r[k]
            sp_sum = sp_sum + _softplus(t)
        cv = jnp.exp(-sp_sum)
        per = cv * (-jnp.log(1.0 - cv + 1e-6))
        rows = i * C_BLK + lax.broadcasted_iota(jnp.int32, (C_BLK, 1), 0)
        per = per * (rows < NC).astype(_F32)
        o_r[...] = jnp.zeros((1, 1, F), _F32) + jnp.sum(per)

    return pl.pallas_call(
        body,
        grid=(C_GRID,),
        in_specs=[
            pl.BlockSpec((KL, C_BLK, 16), lambda i: (0, i, 0)),
            pl.BlockSpec((KL, C_BLK, 1), lambda i: (0, i, 0)),
        ],
        out_specs=pl.BlockSpec((1, 1, F), lambda i: (i, 0, 0)),
        out_shape=jax.ShapeDtypeStruct((C_GRID, 1, F), _F32),
    )(lg3, sign3)


def _sc_gather(table, idx3, d):
    mesh = plsc.VectorSubcoreMesh(core_axis_name="c", subcore_axis_name="s")

    @functools.partial(
        pl.kernel,
        mesh=mesh,
        out_type=jax.ShapeDtypeStruct((B_PAD, d), _F32),
        compiler_params=pltpu.CompilerParams(use_tc_tiling_on_sc=False),
        scratch_types=[
            pltpu.VMEM((N_CH, CHUNK), jnp.int32),
            pltpu.VMEM((CHUNK, d), _F32),
            pltpu.SemaphoreType.DMA,
        ],
    )
    def k(table_hbm, idx_hbm, out_hbm, idx_v, rows_v, sem):
        c = lax.axis_index("c")
        s = lax.axis_index("s")
        wid = s * SC_CORES + c
        base = wid * PER_W
        pltpu.sync_copy(idx_hbm.at[wid], idx_v)

        def body(j, carry):
            pltpu.async_copy(table_hbm.at[idx_v.at[j]], rows_v, sem).wait()
            pltpu.sync_copy(rows_v, out_hbm.at[pl.ds(base + j * CHUNK, CHUNK)])
            return carry

        lax.fori_loop(0, N_CH, body, 0)

    return k(table, idx3)


def _sc_scatter_add(values, idx3, d, zrows):
    mesh = plsc.VectorSubcoreMesh(core_axis_name="c", subcore_axis_name="s")

    @functools.partial(
        pl.kernel,
        mesh=mesh,
        out_type=jax.ShapeDtypeStruct((SC_CORES, NV_PAD, d), _F32),
        compiler_params=pltpu.CompilerParams(use_tc_tiling_on_sc=False),
        scratch_types=[
            pltpu.VMEM_SHARED((NV_PAD, d), _F32),
            pltpu.VMEM((N_CH, CHUNK), jnp.int32),
            pltpu.VMEM((CHUNK, d), _F32),
            pltpu.VMEM((ZROWS, d), _F32),
        ],
    )
    def k(val_hbm, idx_hbm, z_hbm, out_hbm, table_sh, idx_v, val_v, zbuf):
        c = lax.axis_index("c")
        s = lax.axis_index("s")
        wid = s * SC_CORES + c
        base = wid * PER_W
        pltpu.sync_copy(z_hbm, zbuf)
        pltpu.sync_copy(zbuf, table_sh.at[pl.ds(s * ZROWS, ZROWS)])
        plsc.subcore_barrier()
        pltpu.sync_copy(idx_hbm.at[wid], idx_v)

        def body(j, carry):
            pltpu.sync_copy(val_hbm.at[pl.ds(base + j * CHUNK, CHUNK)], val_v)
            pltpu.sync_copy(val_v, table_sh.at[idx_v.at[j]], add=True)
            return carry

        lax.fori_loop(0, N_CH, body, 0)
        plsc.subcore_barrier()
        pltpu.sync_copy(table_sh.at[pl.ds(s * ZROWS, ZROWS)], zbuf)
        pltpu.sync_copy(zbuf, out_hbm.at[c, pl.ds(s * ZROWS, ZROWS)])

    return k(values, idx3, zrows)


def kernel(params, lits_var, lits_sign):
    (wq1, bq1), (wq2, bq2), (wq3, bq3) = params["variables_query"]
    (wc1, bc1), (wc2, bc2), (wc3, bc3) = params["clause_mlp"]
    (wu1, bu1), (wu2, bu2), (wu3, bu3) = params["update_gate"]
    (wo1, bo1), (wo2, bo2), (wo3, bo3) = params["variables_output"]

    wq1v, wq1n = wq1[:F], wq1[F:]
    wc1a, wc1b = wc1[:F], wc1[F:]
    wc3v, wc3n = wc3[:, :Q], wc3[:, Q:]
    bc3v, bc3n = bc3[:Q], bc3[Q:]
    wu1v, wu1g, wu1pn = wu1[:F], wu1[F:F + Q], wu1[F + Q:]
    r1 = lambda b: b.reshape(1, -1)
    bq1, bq2, bq3 = r1(bq1), r1(bq2), r1(bq3)
    bc1, bc2 = r1(bc1), r1(bc2)
    bc3v, bc3n = r1(bc3v), r1(bc3n)
    bu1, bu2, bu3 = r1(bu1), r1(bu2), r1(bu3)
    bo1, bo2, bo3 = r1(bo1), r1(bo2), r1(bo3)

    sign_t = jnp.zeros((KL, NC_PAD, 1), _F32).at[:, :NC, 0].set(lits_sign.T)
    posw = (sign_t > 0).astype(_F32)
    negw = (sign_t < 0).astype(_F32)
    idx_pad = jnp.zeros((KL, NC_PAD), jnp.int32).at[:, :NC].set(lits_var.T)
    idx3 = idx_pad.reshape(SC_W, N_CH, CHUNK)

    base_key = jax.random.key(1234)
    noise = [jax.random.normal(jax.random.fold_in(base_key, s), (NV, 4), _F32)
             for s in range(2)]
    z32 = jnp.zeros((ZROWS, Q), _F32)
    z64 = jnp.zeros((ZROWS, 2 * Q), _F32)

    vars_cur = None
    ncv0 = ps0 = pq0 = None
    loss_partials = []
    logits = None
    for step in range(2):
        q = _qmlp(vars_cur, noise[step], wq1v, wq1n, bq1, wq2, bq2, wq3, bq3)
        gq = _sc_gather(q, idx3, Q).reshape(KL, NC_PAD, Q)
        cl, coeff = _edge_pre(gq, sign_t)
        grad_p = _sc_scatter_add(coeff.reshape(B_PAD, Q), idx3, Q, z32)
        if step == 0:
            evl, ncv0, ps0, pq0 = _clause_mlp_r0(
                cl, posw, negw, wc1a, wc1b, bc1, wc2, bc2,
                wc3v, bc3v, wc3n, bc3n)
        else:
            evl = _clause_mlp_r1(
                cl, ncv0, ps0, pq0, posw, negw,
                wc1a, wc1b, bc1, wc2, bc2, wc3v, bc3v)
        vl_p = _sc_scatter_add(evl.reshape(B_PAD, 2 * Q), idx3, 2 * Q, z64)
        y, vs, vq = _update_mlp(
            vars_cur, grad_p[0, :NV], grad_p[1, :NV],
            vl_p[0, :NV], vl_p[1, :NV],
            wu1v, wu1g, wu1pn, bu1, wu2, bu2, wu3, bu3)
        vars_new, logits, ltab = _apply_logits(
            y, vars_cur, vs, vq, wo1, bo1, wo2, bo2, wo3, bo3)
        lg = _sc_gather(ltab, idx3, 16).reshape(KL, NC_PAD, 16)
        loss_partials.append(_loss_reduce(lg, sign_t))
        vars_cur = vars_new

    pg0 = jnp.sum(loss_partials[0]) / jnp.float32(F)
    pg1 = jnp.sum(loss_partials[1]) / jnp.float32(F)
    total = (jnp.sqrt(pg0 + 1e-6) + jnp.sqrt(pg1 + 1e-6)) / 2.0
    return logits, total


# speedup vs baseline: 3.7353x; 1.0758x over previous
import functools

import jax
import jax.numpy as jnp
from jax import lax
from jax.experimental import pallas as pl
from jax.experimental.pallas import tpu as pltpu
from jax.experimental.pallas import tpu_sc as plsc

NV = 10000
NC = 42000
KL = 3
F = 128
Q = 32

NC_PAD = 45056
C_BLK = 2048
C_GRID = NC_PAD // C_BLK
B_PAD = KL * NC_PAD
V_BLK = 2000
V_GRID = NV // V_BLK

SC_CORES = 2
SC_SUB = 16
SC_W = SC_CORES * SC_SUB
PER_W = B_PAD // SC_W
CHUNK = 128
N_CH = PER_W // CHUNK
NV_PAD = 10240
ZROWS = NV_PAD // SC_SUB

_F32 = jnp.float32


def _relu(x):
    return jnp.maximum(x, 0.0)


def _dot(a, b):
    return jnp.dot(a.astype(jnp.bfloat16), b.astype(jnp.bfloat16),
                   preferred_element_type=_F32)


def _softplus(x):
    return jnp.maximum(x, 0.0) + jnp.log1p(jnp.exp(-jnp.abs(x)))


def _sigmoid(x):
    return 1.0 / (1.0 + jnp.exp(-x))


def _const_state_row():
    import numpy as np
    lane = lax.broadcasted_iota(jnp.int32, (1, F), 1)
    sq = np.float32(np.sqrt(128.0))
    hi = np.float32(np.float32(np.float32(1.0 - 1.0 / 128.0) * sq)
                    * np.float32(0.25))
    lo = np.float32(np.float32(np.float32(-1.0 / 128.0) * sq)
                    * np.float32(0.25))
    return jnp.where(lane == 0, jnp.float32(hi), jnp.float32(lo))


def _full_spec(shape):
    nd = len(shape)
    return pl.BlockSpec(shape, lambda *_: (0,) * nd)


def _qmlp(vars_in, noise, w1, b1, w2, b2, w3, b3):
    has_v = vars_in is not None

    def body(*refs):
        if has_v:
            v_r, n_r, w1_r, b1_r, w2_r, b2_r, w3_r, b3_r, o_r = refs
            xv = v_r[...]
        else:
            n_r, w1_r, b1_r, w2_r, b2_r, w3_r, b3_r, o_r = refs
            xv = jnp.broadcast_to(_const_state_row(), (V_BLK, F))
        x = jnp.concatenate([xv, n_r[...]], axis=1)
        h = _relu(_dot(x, w1_r[...]) + b1_r[...])
        h = _relu(_dot(h, w2_r[...]) + b2_r[...])
        o_r[...] = _dot(h, w3_r[...]) + b3_r[...]

    in_specs = []
    args = []
    if has_v:
        in_specs.append(pl.BlockSpec((V_BLK, F), lambda i: (i, 0)))
        args.append(vars_in)
    in_specs += [pl.BlockSpec((V_BLK, 4), lambda i: (i, 0))]
    args += [noise]
    for w in (w1, b1, w2, b2, w3, b3):
        in_specs.append(_full_spec(w.shape))
        args.append(w)
    return pl.pallas_call(
        body,
        grid=(V_GRID,),
        in_specs=in_specs,
        out_specs=pl.BlockSpec((V_BLK, Q), lambda i: (i, 0)),
        out_shape=jax.ShapeDtypeStruct((NV, Q), _F32),
    )(*args)


def _edge_pre(g3, sign3):
    def body(g_r, s_r, cl_r, co_r):
        t = []
        sp = []
        sp_sum = None
        for k in range(KL):
            tk = g_r[k] * s_r[k]
            t.append(tk)
            spk = _softplus(tk)
            sp.append(spk)
            sp_sum = spk if sp_sum is None else sp_sum + spk
        cl = jnp.exp(-sp_sum)
        cl_r[...] = cl
        ncl = -cl
        for k in range(KL):
            co_r[k] = (ncl * jnp.exp(t[k] - sp[k])) * s_r[k]

    return pl.pallas_call(
        body,
        grid=(C_GRID,),
        in_specs=[
            pl.BlockSpec((KL, C_BLK, Q), lambda i: (0, i, 0)),
            pl.BlockSpec((KL, C_BLK, 1), lambda i: (0, i, 0)),
        ],
        out_specs=[
            pl.BlockSpec((C_BLK, Q), lambda i: (i, 0)),
            pl.BlockSpec((KL, C_BLK, Q), lambda i: (0, i, 0)),
        ],
        out_shape=[
            jax.ShapeDtypeStruct((NC_PAD, Q), _F32),
            jax.ShapeDtypeStruct((KL, NC_PAD, Q), _F32),
        ],
    )(g3, sign3)


def _clause_mlp_r0(cl, padm, w1, b1, w2, b2, w3v, b3v, w3n, b3n):
    def body(cl_r, pm_r, w1_r, b1_r, w2_r, b2_r, w3v_r, b3v_r,
             w3n_r, b3n_r, evl_r, ncv_r, ps_r, pq_r):
        i = pl.program_id(0)
        cs = jnp.broadcast_to(_const_state_row(), (C_BLK, F))
        x = jnp.concatenate([cs, cl_r[...]], axis=1)
        h = _relu(_dot(x, w1_r[...]) + b1_r[...])
        h = _relu(_dot(h, w2_r[...]) + b2_r[...])
        vla = _dot(h, w3v_r[...]) + b3v_r[...]
        ncv = _dot(h, w3n_r[...]) + b3n_r[...]
        ncv_r[...] = ncv
        for k in range(KL):
            evl_r[k] = vla * pm_r[k]
        rows = i * C_BLK + lax.broadcasted_iota(jnp.int32, (C_BLK, 1), 0)
        m = (rows < NC).astype(_F32)
        ncv_m = ncv * m
        ps_r[...] = jnp.sum(ncv_m, axis=0).reshape(1, 1, F)
        pq_r[...] = jnp.zeros((1, 1, F), _F32) + jnp.sum(ncv_m * ncv_m)

    return pl.pallas_call(
        body,
        grid=(C_GRID,),
        in_specs=[
            pl.BlockSpec((C_BLK, Q), lambda i: (i, 0)),
            pl.BlockSpec((KL, C_BLK, 1), lambda i: (0, i, 0)),
            _full_spec(w1.shape), _full_spec(b1.shape),
            _full_spec(w2.shape), _full_spec(b2.shape),
            _full_spec(w3v.shape), _full_spec(b3v.shape),
            _full_spec(w3n.shape), _full_spec(b3n.shape),
        ],
        out_specs=[
            pl.BlockSpec((KL, C_BLK, Q), lambda i: (0, i, 0)),
            pl.BlockSpec((C_BLK, F), lambda i: (i, 0)),
            pl.BlockSpec((1, 1, F), lambda i: (i, 0, 0)),
            pl.BlockSpec((1, 1, F), lambda i: (i, 0, 0)),
        ],
        out_shape=[
            jax.ShapeDtypeStruct((KL, NC_PAD, Q), _F32),
            jax.ShapeDtypeStruct((NC_PAD, F), _F32),
            jax.ShapeDtypeStruct((C_GRID, 1, F), _F32),
            jax.ShapeDtypeStruct((C_GRID, 1, F), _F32),
        ],
    )(cl, padm, w1, b1, w2, b2, w3v, b3v, w3n, b3n)


def _clause_mlp_r1(cl, ncv0, ps0, pq0, padm,
                   w1, b1, w2, b2, w3v, b3v):
    def body(cl_r, ncv_r, ps_r, pq_r, pm_r, w1_r, b1_r,
             w2_r, b2_r, w3v_r, b3v_r, evl_r):
        mu = jnp.sum(ps_r[...], axis=0).reshape(1, F) / jnp.float32(NC)
        m2 = jnp.sum(mu * mu)
        tot = jnp.sum(pq_r[...]) / jnp.float32(F)
        scale = lax.rsqrt((tot - jnp.float32(NC) * m2) / jnp.float32(NC)
                          + jnp.float32(1e-6))
        crow = _const_state_row()
        cs = ((ncv_r[...] - mu) * scale) * 0.25 + 0.1 * crow
        cs = 0.2 * cs + 0.8 * cs
        x = jnp.concatenate([cs, cl_r[...]], axis=1)
        h = _relu(_dot(x, w1_r[...]) + b1_r[...])
        h = _relu(_dot(h, w2_r[...]) + b2_r[...])
        vla = _dot(h, w3v_r[...]) + b3v_r[...]
        for k in range(KL):
            evl_r[k] = vla * pm_r[k]

    return pl.pallas_call(
        body,
        grid=(C_GRID,),
        in_specs=[
            pl.BlockSpec((C_BLK, Q), lambda i: (i, 0)),
            pl.BlockSpec((C_BLK, F), lambda i: (i, 0)),
            _full_spec(ps0.shape), _full_spec(pq0.shape),
            pl.BlockSpec((KL, C_BLK, 1), lambda i: (0, i, 0)),
            _full_spec(w1.shape), _full_spec(b1.shape),
            _full_spec(w2.shape), _full_spec(b2.shape),
            _full_spec(w3v.shape), _full_spec(b3v.shape),
        ],
        out_specs=pl.BlockSpec((KL, C_BLK, Q), lambda i: (0, i, 0)),
        out_shape=jax.ShapeDtypeStruct((KL, NC_PAD, Q), _F32),
    )(cl, ncv0, ps0, pq0, padm, w1, b1, w2, b2, w3v, b3v)


def _update_mlp(vars_in, g0, g1, p0, p1, n0, n1,
                w1, b1, w2, b2, w3, b3):
    has_v = vars_in is not None

    def body(*refs):
        if has_v:
            (v_r, g0_r, g1_r, p0_r, p1_r, n0_r, n1_r,
             w1_r, b1_r, w2_r, b2_r, w3_r, b3_r, y_r, ps_r, pq_r) = refs
            xv = v_r[...]
        else:
            (g0_r, g1_r, p0_r, p1_r, n0_r, n1_r,
             w1_r, b1_r, w2_r, b2_r, w3_r, b3_r, y_r, ps_r, pq_r) = refs
            xv = jnp.broadcast_to(_const_state_row(), (V_BLK, F))
        grad = g0_r[...] + g1_r[...]
        vlp = p0_r[...] + p1_r[...]
        vln = n0_r[...] + n1_r[...]
        x = jnp.concatenate([xv, grad, vlp, vln], axis=1)
        h = _relu(_dot(x, w1_r[...]) + b1_r[...])
        h = _relu(_dot(h, w2_r[...]) + b2_r[...])
        y = _dot(h, w3_r[...]) + b3_r[...]
        y_r[...] = y
        ps_r[...] = jnp.sum(y, axis=0).reshape(1, 1, F)
        pq_r[...] = jnp.zeros((1, 1, F), _F32) + jnp.sum(y * y)

    in_specs = []
    args = []
    if has_v:
        in_specs.append(pl.BlockSpec((V_BLK, F), lambda i: (i, 0)))
        args.append(vars_in)
    in_specs += [pl.BlockSpec((V_BLK, Q), lambda i: (i, 0))] * 6
    args += [g0, g1, p0, p1, n0, n1]
    for w in (w1, b1, w2, b2, w3, b3):
        in_specs.append(_full_spec(w.shape))
        args.append(w)
    return pl.pallas_call(
        body,
        grid=(V_GRID,),
        in_specs=in_specs,
        out_specs=[
            pl.BlockSpec((V_BLK, F), lambda i: (i, 0)),
            pl.BlockSpec((1, 1, F), lambda i: (i, 0, 0)),
            pl.BlockSpec((1, 1, F), lambda i: (i, 0, 0)),
        ],
        out_shape=[
            jax.ShapeDtypeStruct((NV, F), _F32),
            jax.ShapeDtypeStruct((V_GRID, 1, F), _F32),
            jax.ShapeDtypeStruct((V_GRID, 1, F), _F32),
        ],
    )(*args)


def _apply_logits(y, prev, ps, pq, w1, b1, w2, b2, w3, b3):
    has_p = prev is not None

    def body(*refs):
        if has_p:
            (y_r, p_r, ps_r, pq_r, w1_r, b1_r, w2_r, b2_r, w3_r, b3_r,
             nv_r, lg_r, lt_r) = refs
            pv = p_r[...]
        else:
            (y_r, ps_r, pq_r, w1_r, b1_r, w2_r, b2_r, w3_r, b3_r,
             nv_r, lg_r, lt_r) = refs
            pv = _const_state_row()
        mu = jnp.sum(ps_r[...], axis=0).reshape(1, F) / jnp.float32(NV)
        m2 = jnp.sum(mu * mu)
        tot = jnp.sum(pq_r[...]) / jnp.float32(F)
        scale = lax.rsqrt((tot - jnp.float32(NV) * m2) / jnp.float32(NV)
                          + jnp.float32(1e-6))
        nv = ((y_r[...] - mu) * scale) * 0.25 + 0.1 * pv
        nv_r[...] = nv * 0.2 + nv * 0.8
        h = _relu(_dot(nv, w1_r[...]) + b1_r[...])
        h = _relu(_dot(h, w2_r[...]) + b2_r[...])
        lg = _dot(h, w3_r[...]) + b3_r[...]
        lg_r[...] = lg
        lt_r[...] = jnp.broadcast_to(lg, (V_BLK, 16))

    in_specs = [pl.BlockSpec((V_BLK, F), lambda i: (i, 0))]
    args = [y]
    if has_p:
        in_specs.append(pl.BlockSpec((V_BLK, F), lambda i: (i, 0)))
        args.append(prev)
    in_specs += [_full_spec(ps.shape), _full_spec(pq.shape)]
    args += [ps, pq]
    for w in (w1, b1, w2, b2, w3, b3):
        in_specs.append(_full_spec(w.shape))
        args.append(w)
    return pl.pallas_call(
        body,
        grid=(V_GRID,),
        in_specs=in_specs,
        out_specs=[
            pl.BlockSpec((V_BLK, F), lambda i: (i, 0)),
            pl.BlockSpec((V_BLK, 1), lambda i: (i, 0)),
            pl.BlockSpec((V_BLK, 16), lambda i: (i, 0)),
        ],
        out_shape=[
            jax.ShapeDtypeStruct((NV, F), _F32),
            jax.ShapeDtypeStruct((NV, 1), _F32),
            jax.ShapeDtypeStruct((NV, 16), _F32),
        ],
    )(*args)


def _loss_reduce(lg3, sign3):
    def body(g_r, s_r, o_r):
        i = pl.program_id(0)
        sp_sum = jnp.zeros((C_BLK, 1), _F32)
        for k in range(KL):
            t = g_r[k][:, 0:1] * s_r[k]
            sp_sum = sp_sum + _softplus(t)
        cv = jnp.exp(-sp_sum)
        per = cv * (-jnp.log(1.0 - cv + 1e-6))
        rows = i * C_BLK + lax.broadcasted_iota(jnp.int32, (C_BLK, 1), 0)
        per = per * (rows < NC).astype(_F32)
        o_r[...] = jnp.zeros((1, 1, F), _F32) + jnp.sum(per)

    return pl.pallas_call(
        body,
        grid=(C_GRID,),
        in_specs=[
            pl.BlockSpec((KL, C_BLK, 16), lambda i: (0, i, 0)),
            pl.BlockSpec((KL, C_BLK, 1), lambda i: (0, i, 0)),
        ],
        out_specs=pl.BlockSpec((1, 1, F), lambda i: (i, 0, 0)),
        out_shape=jax.ShapeDtypeStruct((C_GRID, 1, F), _F32),
    )(lg3, sign3)


G_CH = 11
N_GRP = N_CH // G_CH
GROWS = G_CH * CHUNK


def _sc_gather(table, idx3, d):
    mesh = plsc.VectorSubcoreMesh(core_axis_name="c", subcore_axis_name="s")

    @functools.partial(
        pl.kernel,
        mesh=mesh,
        out_type=jax.ShapeDtypeStruct((B_PAD, d), _F32),
        compiler_params=pltpu.CompilerParams(use_tc_tiling_on_sc=False),
        scratch_types=[
            pltpu.VMEM((N_CH, CHUNK), jnp.int32),
            pltpu.VMEM((2, GROWS, d), _F32),
            pltpu.SemaphoreType.DMA,
            pltpu.SemaphoreType.DMA,
            pltpu.SemaphoreType.DMA,
            pltpu.SemaphoreType.DMA,
        ],
    )
    def k(table_hbm, idx_hbm, out_hbm, idx_v, rows_v, sg0, sg1, so0, so1):
        c = lax.axis_index("c")
        s = lax.axis_index("s")
        wid = s * SC_CORES + c
        base = wid * PER_W
        pltpu.sync_copy(idx_hbm.at[wid], idx_v)
        sg = [sg0, sg1]
        so = [so0, so1]

        def start_gather(g, buf):
            descs = []
            for b in range(G_CH):
                descs.append(pltpu.async_copy(
                    table_hbm.at[idx_v.at[g * G_CH + b]],
                    rows_v.at[buf].at[pl.ds(b * CHUNK, CHUNK)],
                    sg[buf]))
            return descs

        def start_out(g, buf):
            return pltpu.async_copy(
                rows_v.at[buf], out_hbm.at[pl.ds(base + g * GROWS, GROWS)],
                so[buf])

        dg0 = start_gather(0, 0)
        dg1 = start_gather(1, 1)
        for dsc in dg0:
            dsc.wait()
        do0 = start_out(0, 0)
        for dsc in dg1:
            dsc.wait()
        do1 = start_out(1, 1)
        do0.wait()
        dg2 = start_gather(2, 0)
        for dsc in dg2:
            dsc.wait()
        do2 = start_out(2, 0)
        do1.wait()
        do2.wait()

    return k(table, idx3)


SG_CH = 3
SG_N = N_CH // SG_CH
SGROWS = SG_CH * CHUNK


def _sc_scatter_add(values, idx3, d, zrows, n_rows):
    mesh = plsc.VectorSubcoreMesh(core_axis_name="c", subcore_axis_name="s")
    rows_per_sub = n_rows // SC_SUB
    n_zch = rows_per_sub // ZROWS

    @functools.partial(
        pl.kernel,
        mesh=mesh,
        out_type=jax.ShapeDtypeStruct((SC_CORES, n_rows, d), _F32),
        compiler_params=pltpu.CompilerParams(use_tc_tiling_on_sc=False),
        scratch_types=[
            pltpu.VMEM_SHARED((n_rows, d), _F32),
            pltpu.VMEM((N_CH, CHUNK), jnp.int32),
            pltpu.VMEM((3, SGROWS, d), _F32),
            pltpu.VMEM((ZROWS, d), _F32),
            pltpu.SemaphoreType.DMA,
            pltpu.SemaphoreType.DMA,
            pltpu.SemaphoreType.DMA,
            pltpu.SemaphoreType.DMA,
            pltpu.SemaphoreType.DMA,
            pltpu.SemaphoreType.DMA,
        ],
    )
    def k(val_hbm, idx_hbm, z_hbm, out_hbm, table_sh, idx_v, val_v, zbuf,
          sv0, sv1, sv2, ss0, ss1, ss2):
        c = lax.axis_index("c")
        s = lax.axis_index("s")
        wid = s * SC_CORES + c
        base = wid * PER_W
        pltpu.sync_copy(z_hbm, zbuf)
        for t in range(n_zch):
            pltpu.sync_copy(
                zbuf, table_sh.at[pl.ds(s * rows_per_sub + t * ZROWS, ZROWS)])
        plsc.subcore_barrier()
        pltpu.sync_copy(idx_hbm.at[wid], idx_v)
        sv = [sv0, sv1, sv2]
        ss = [ss0, ss1, ss2]

        def start_val(g):
            return pltpu.async_copy(
                val_hbm.at[pl.ds(base + g * SGROWS, SGROWS)],
                val_v.at[g % 3], sv[g % 3])

        def start_scat(g):
            descs = []
            for b in range(SG_CH):
                descs.append(pltpu.async_copy(
                    val_v.at[g % 3].at[pl.ds(b * CHUNK, CHUNK)],
                    table_sh.at[idx_v.at[g * SG_CH + b]],
                    ss[g % 3], add=True))
            return descs

        dval = {0: start_val(0), 1: start_val(1)}
        dscat = {}
        for g in range(SG_N):
            dval[g].wait()
            dscat[g] = start_scat(g)
            if g - 1 in dscat:
                for dsc in dscat.pop(g - 1):
                    dsc.wait()
            if g + 2 < SG_N:
                dval[g + 2] = start_val(g + 2)
        for dsc in dscat.pop(SG_N - 1):
            dsc.wait()
        plsc.subcore_barrier()
        for t in range(n_zch):
            pltpu.sync_copy(
                table_sh.at[pl.ds(s * rows_per_sub + t * ZROWS, ZROWS)], zbuf)
            pltpu.sync_copy(
                zbuf, out_hbm.at[c, pl.ds(s * rows_per_sub + t * ZROWS, ZROWS)])

    return k(values, idx3, zrows)


def kernel(params, lits_var, lits_sign):
    (wq1, bq1), (wq2, bq2), (wq3, bq3) = params["variables_query"]
    (wc1, bc1), (wc2, bc2), (wc3, bc3) = params["clause_mlp"]
    (wu1, bu1), (wu2, bu2), (wu3, bu3) = params["update_gate"]
    (wo1, bo1), (wo2, bo2), (wo3, bo3) = params["variables_output"]

    wc3v, wc3n = wc3[:, :Q], wc3[:, Q:]
    bc3v, bc3n = bc3[:Q], bc3[Q:]
    r1 = lambda b: b.reshape(1, -1)
    bq1, bq2, bq3 = r1(bq1), r1(bq2), r1(bq3)
    bc1, bc2 = r1(bc1), r1(bc2)
    bc3v, bc3n = r1(bc3v), r1(bc3n)
    bu1, bu2, bu3 = r1(bu1), r1(bu2), r1(bu3)
    bo1, bo2, bo3 = r1(bo1), r1(bo2), r1(bo3)

    sign_t = jnp.zeros((KL, NC_PAD, 1), _F32).at[:, :NC, 0].set(lits_sign.T)
    padm = (sign_t != 0).astype(_F32)
    idx_pad = jnp.zeros((KL, NC_PAD), jnp.int32).at[:, :NC].set(lits_var.T)
    idx3 = idx_pad.reshape(SC_W, N_CH, CHUNK)
    idx_vl = idx_pad + NV_PAD * (sign_t[:, :, 0] < 0).astype(jnp.int32)
    idx_vl3 = idx_vl.reshape(SC_W, N_CH, CHUNK)

    base_key = jax.random.key(1234)
    noise = [jax.random.normal(jax.random.fold_in(base_key, s), (NV, 4), _F32)
             for s in range(2)]
    z32 = jnp.zeros((ZROWS, Q), _F32)

    vars_cur = None
    ncv0 = ps0 = pq0 = None
    loss_partials = []
    logits = None
    for step in range(2):
        q = _qmlp(vars_cur, noise[step], wq1, bq1, wq2, bq2, wq3, bq3)
        gq = _sc_gather(q, idx3, Q).reshape(KL, NC_PAD, Q)
        cl, coeff = _edge_pre(gq, sign_t)
        grad_p = _sc_scatter_add(coeff.reshape(B_PAD, Q), idx3, Q, z32,
                                 NV_PAD)
        if step == 0:
            evl, ncv0, ps0, pq0 = _clause_mlp_r0(
                cl, padm, wc1, bc1, wc2, bc2,
                wc3v, bc3v, wc3n, bc3n)
        else:
            evl = _clause_mlp_r1(
                cl, ncv0, ps0, pq0, padm,
                wc1, bc1, wc2, bc2, wc3v, bc3v)
        vl_p = _sc_scatter_add(evl.reshape(B_PAD, Q), idx_vl3, Q, z32,
                               2 * NV_PAD)
        y, vs, vq = _update_mlp(
            vars_cur, grad_p[0, :NV], grad_p[1, :NV],
            vl_p[0, :NV], vl_p[1, :NV],
            vl_p[0, NV_PAD:NV_PAD + NV], vl_p[1, NV_PAD:NV_PAD + NV],
            wu1, bu1, wu2, bu2, wu3, bu3)
        vars_new, logits, ltab = _apply_logits(
            y, vars_cur, vs, vq, wo1, bo1, wo2, bo2, wo3, bo3)
        lg = _sc_gather(ltab, idx3, 16).reshape(KL, NC_PAD, 16)
        loss_partials.append(_loss_reduce(lg, sign_t))
        vars_cur = vars_new

    pg0 = jnp.sum(loss_partials[0]) / jnp.float32(F)
    pg1 = jnp.sum(loss_partials[1]) / jnp.float32(F)
    total = (jnp.sqrt(pg0 + 1e-6) + jnp.sqrt(pg1 + 1e-6)) / 2.0
    return logits, total
